# baseline (speedup 1.0000x reference)
import jax
import jax.numpy as jnp
from jax import lax
from jax.experimental import pallas as pl
from jax.experimental.pallas import tpu as pltpu
from jax.experimental.pallas import tpu_sc as plsc

B = 16384
DIM = 64
N = 1000000
NC = 2
NS = 16
NW = NC * NS
CPW = B // NW
NGRP = CPW // 16

NTC = (N + 127) // 128
NFULL = (N // 128) * 128
TC_LO = NTC // NW
TC_XTRA = NTC - TC_LO * NW
CHW = 640
NCHUNK = 50
WCAP = 1600
CCAP = 112
SENT = 2147483647

_LN2 = 0.6931471805599453
_SQRT2 = 1.4142135623730951


def _iota16():
    return lax.iota(jnp.int32, 16)


def _splat(x, dtype=jnp.int32):
    return jnp.zeros((16,), dtype) + x


def _ln(x):
    bits = plsc.bitcast(x, jnp.int32)
    ex = lax.shift_right_logical(bits, 23) - 127
    m = plsc.bitcast(
        jnp.bitwise_or(jnp.bitwise_and(bits, 0x7FFFFF), 0x3F800000),
        jnp.float32,
    )
    big = m > _SQRT2
    m = jnp.where(big, m * 0.5, m)
    exf = ex.astype(jnp.float32) + jnp.where(big, 1.0, 0.0)
    z = (m - 1.0) / (m + 1.0)
    z2 = z * z
    p = 1.0 + z2 * (1.0 / 3.0 + z2 * (1.0 / 5.0 + z2 * (1.0 / 7.0)))
    return exf * _LN2 + 2.0 * z * p


def _k1_body(et_h, left_h, right_h, rows_h,
             piece, wl_i, wl_o, cbuf0, cbuf1, tailbuf,
             ci0, co0, ci1, co1, stage0, stage1, semf0, semf1, sems):
    c = lax.axis_index("c")
    s = lax.axis_index("s")
    wid = s * NC + c
    ts = wid * TC_LO + jnp.minimum(wid, TC_XTRA)
    te = ts + TC_LO + jnp.where(wid < TC_XTRA, 1, 0)
    rs = ts * 128
    re = jnp.minimum(te * 128, N)
    rem_end = jnp.minimum(re, NFULL)

    def _fetch_descs(g, buf, semf):
        fb = pl.multiple_of(
            jnp.minimum(rs + g * CHW, rem_end - CHW), 128)
        return [
            pltpu.make_async_copy(
                et_h.at[pl.ds(8 * a, 8), pl.ds(fb, CHW)],
                buf.at[pl.ds(8 * a, 8), :], semf)
            for a in range(8)
        ]

    class _FetchGroup:
        def __init__(self, descs):
            self.descs = descs

        def start(self):
            for d in self.descs:
                d.start()

        def wait(self):
            for d in self.descs:
                d.wait()

    def _fetch_desc(g, buf, semf):
        return _FetchGroup(_fetch_descs(g, buf, semf))

    _fetch_desc(0, cbuf0, semf0).start()
    _fetch_desc(1, cbuf1, semf1).start()

    def _filter(src_h, side, cnt0):
        def piece_loop(p, cnt):
            pltpu.sync_copy(src_h.at[pl.ds(p * 2048, 2048)], piece)

            def t_loop(t, carry):
                cnt, offv = carry
                v = piece[pl.ds(t * 16, 16)]
                mask = (v >= rs) & (v < re)
                pos = plsc.cumsum(mask.astype(jnp.int32))
                idxs = pos + (cnt - 1)
                plsc.store_scatter(wl_i, [idxs], v, mask=mask)
                plsc.store_scatter(wl_o, [idxs], offv, mask=mask)
                return cnt + pos[15], offv + 16 * DIM
            offv0 = (_splat(side * B + p * 2048) + _iota16()) * DIM
            cnt, _ = lax.fori_loop(0, 128, t_loop, (cnt, offv0))
            return cnt
        return lax.fori_loop(0, 8, piece_loop, cnt0)

    m = _filter(left_h, 0, jnp.int32(0))
    m = _filter(right_h, 1, m)
    plsc.store_scatter(wl_i, [_iota16() + m], _splat(SENT))

    def _refilter(g, ci, co):
        lo = rs + g * CHW
        hi = jnp.minimum(lo + CHW, rem_end)
        return _refilter_win(lo, hi, ci, co)

    def _refilter_win(lo, hi, ci, co):
        trips = (m + 15) // 16

        def t_loop(t, mc):
            v = wl_i[pl.ds(t * 16, 16)]
            ov = wl_o[pl.ds(t * 16, 16)]
            mask = (v >= lo) & (v < hi)
            pos = plsc.cumsum(mask.astype(jnp.int32))
            idxs = pos + (mc - 1)
            plsc.store_scatter(ci, [idxs], v, mask=mask)
            plsc.store_scatter(co, [idxs], ov, mask=mask)
            return mc + pos[15]
        mc = lax.fori_loop(0, trips, t_loop, jnp.int32(0))
        c0 = ci[pl.ds(0, 16)]
        o0 = co[pl.ds(0, 16)]
        plsc.store_scatter(ci, [_iota16() + mc], _splat(c0[0]))
        plsc.store_scatter(co, [_iota16() + mc], _splat(o0[0]))
        return mc

    def _process(buf, g, mc, ci, co, stage):
        fb = pl.multiple_of(
            jnp.minimum(rs + g * CHW, rem_end - CHW), 128)
        ngr = (mc + 15) // 16

        def g_loop(gr, _):
            clv = ci[pl.ds(gr * 16, 16)] - fb
            bov = co[pl.ds(gr * 16, 16)]
            for i in range(16):
                cl = clv[i]
                slot = gr * 16 + i
                for q in range(4):
                    vq = plsc.load_gather(
                        buf, [_iota16() + 16 * q, _splat(cl)])
                    stage[pl.ds(slot * DIM + 16 * q, 16)] = vq
                pltpu.async_copy(
                    stage.at[pl.ds(slot * DIM, DIM)],
                    rows_h.at[pl.ds(pl.multiple_of(bov[i], DIM), DIM)],
                    sems)
            return 0
        lax.fori_loop(0, ngr, g_loop, 0)
        return ngr * 16

    def _drain(n):
        def d_loop(i, _):
            pltpu.make_async_copy(
                stage0.at[pl.ds(0, DIM)],
                rows_h.at[pl.ds(0, DIM)], sems).wait()
            return 0
        lax.fori_loop(0, n, d_loop, 0)

    mc_first = _refilter(0, ci0, co0)

    def outer(gg, carry):
        prev_n, mc0 = carry
        g0 = gg * 2
        mc1 = _refilter(g0 + 1, ci1, co1)
        _fetch_desc(g0, cbuf0, semf0).wait()
        n0 = _process(cbuf0, g0, mc0, ci0, co0, stage0)

        @pl.when(g0 + 2 < NCHUNK)
        def _():
            _fetch_desc(g0 + 2, cbuf0, semf0).start()
        _drain(prev_n)
        mc0n = _refilter(g0 + 2, ci0, co0)
        _fetch_desc(g0 + 1, cbuf1, semf1).wait()
        n1 = _process(cbuf1, g0 + 1, mc1, ci1, co1, stage1)

        @pl.when(g0 + 3 < NCHUNK)
        def _():
            _fetch_desc(g0 + 3, cbuf1, semf1).start()
        _drain(n0)
        return n1, mc0n

    last_n, _ = lax.fori_loop(
        0, NCHUNK // 2, outer, (jnp.int32(0), mc_first))
    _drain(last_n)

    @pl.when(wid == NW - 1)
    def _():
        pltpu.sync_copy(et_h.at[:, pl.ds(NFULL, N - NFULL)], tailbuf)
        mc = _refilter_win(NFULL, N, ci0, co0)
        ngr = (mc + 15) // 16

        def g_loop(gr, _):
            clv = ci0[pl.ds(gr * 16, 16)] - NFULL
            bov = co0[pl.ds(gr * 16, 16)]
            for i in range(16):
                cl = clv[i]
                slot = gr * 16 + i
                for q in range(4):
                    vq = plsc.load_gather(
                        tailbuf, [_iota16() + 16 * q, _splat(cl)])
                    stage0[pl.ds(slot * DIM + 16 * q, 16)] = vq
                pltpu.async_copy(
                    stage0.at[pl.ds(slot * DIM, DIM)],
                    rows_h.at[pl.ds(pl.multiple_of(bov[i], DIM), DIM)],
                    sems)
            return 0
        lax.fori_loop(0, ngr, g_loop, 0)
        _drain(ngr * 16)


def _k1(et, left, right):
    mesh = plsc.VectorSubcoreMesh(
        core_axis_name="c", subcore_axis_name="s",
        num_cores=NC, num_subcores=NS)
    call = pl.kernel(
        _k1_body,
        out_type=jax.ShapeDtypeStruct((2 * B * DIM,), jnp.float32),
        mesh=mesh,
        scratch_types=[
            pltpu.VMEM((2048,), jnp.int32),
            pltpu.VMEM((WCAP + 16,), jnp.int32),
            pltpu.VMEM((WCAP + 16,), jnp.int32),
            pltpu.VMEM((DIM, CHW), jnp.float32),
            pltpu.VMEM((DIM, CHW), jnp.float32),
            pltpu.VMEM((DIM, N - NFULL), jnp.float32),
            pltpu.VMEM((CCAP + 32,), jnp.int32),
            pltpu.VMEM((CCAP + 32,), jnp.int32),
            pltpu.VMEM((CCAP + 32,), jnp.int32),
            pltpu.VMEM((CCAP + 32,), jnp.int32),
            pltpu.VMEM(((CCAP + 16) * DIM,), jnp.float32),
            pltpu.VMEM(((CCAP + 16) * DIM,), jnp.float32),
            pltpu.SemaphoreType.DMA,
            pltpu.SemaphoreType.DMA,
            pltpu.SemaphoreType.DMA,
        ],
        compiler_params=pltpu.CompilerParams(
            needs_layout_passes=False, use_tc_tiling_on_sc=True),
    )
    return call(et, left, right)


def _k2_body(rows_h, bias_h, rew_h, right_h, out_h,
             ridx, lbuf, rbuf, bvals, rew, sbuf, outbuf, sem):
    c = lax.axis_index("c")
    s = lax.axis_index("s")
    wid = s * NC + c
    base = wid * CPW

    for j in range(CPW // 128):
        pltpu.sync_copy(right_h.at[pl.ds(base + j * 128, 128)], ridx.at[j])
    cps = [
        pltpu.async_copy(
            rows_h.at[pl.ds(base * DIM, CPW * DIM)], lbuf, sem),
        pltpu.async_copy(
            rows_h.at[pl.ds((B + base) * DIM, CPW * DIM)], rbuf, sem),
    ]
    for j in range(CPW // 128):
        cps.append(pltpu.async_copy(
            bias_h.at[ridx.at[j]], bvals.at[pl.ds(j * 128, 128)], sem))
    pltpu.sync_copy(rew_h.at[pl.ds(base, CPW)], rew)
    for cp in cps:
        cp.wait()

    def grp(g, acc):
        for i in range(16):
            r = g * 16 + i
            a = (lbuf[pl.ds(r * DIM, 16)] * rbuf[pl.ds(r * DIM, 16)]
                 + lbuf[pl.ds(r * DIM + 16, 16)] * rbuf[pl.ds(r * DIM + 16, 16)]
                 + lbuf[pl.ds(r * DIM + 32, 16)] * rbuf[pl.ds(r * DIM + 32, 16)]
                 + lbuf[pl.ds(r * DIM + 48, 16)] * rbuf[pl.ds(r * DIM + 48, 16)])
            sbuf[pl.ds(i * 16, 16)] = a
        rowbase = _iota16() * 16
        sc = jnp.zeros((16,), jnp.float32)
        for j in range(16):
            sc = sc + plsc.load_gather(sbuf, [rowbase + j])
        score = sc + bvals[pl.ds(g * 16, 16)]
        prob = 1.0 / (1.0 + jnp.exp(-score))
        prob = jnp.minimum(jnp.maximum(prob, 1e-5), 1.0 - 1e-5)
        return acc + _ln(prob) * _ln(1.0 - rew[pl.ds(g * 16, 16)])

    acc = lax.fori_loop(0, NGRP, grp, jnp.zeros((16,), jnp.float32))
    outbuf[...] = acc
    pltpu.sync_copy(outbuf, out_h.at[wid])


def _k2(rows, bias, reward, right):
    mesh = plsc.VectorSubcoreMesh(
        core_axis_name="c", subcore_axis_name="s",
        num_cores=NC, num_subcores=NS)
    call = pl.kernel(
        _k2_body,
        out_type=jax.ShapeDtypeStruct((NW, 16), jnp.float32),
        mesh=mesh,
        scratch_types=[
            pltpu.VMEM((CPW // 128, 128), jnp.int32),
            pltpu.VMEM((CPW * DIM,), jnp.float32),
            pltpu.VMEM((CPW * DIM,), jnp.float32),
            pltpu.VMEM((CPW,), jnp.float32),
            pltpu.VMEM((CPW,), jnp.float32),
            pltpu.VMEM((256,), jnp.float32),
            pltpu.VMEM((16,), jnp.float32),
            pltpu.SemaphoreType.DMA,
        ],
        compiler_params=pltpu.CompilerParams(
            needs_layout_passes=False, use_tc_tiling_on_sc=False),
    )
    return call(rows, bias, reward, right)


@jax.jit
def kernel(emb, bias, reward, left, right):
    left = left.astype(jnp.int32)
    right = right.astype(jnp.int32)
    rows = _k1(emb.T, left, right)
    part = _k2(rows, bias, reward, right)
    return jnp.sum(part)


# speedup vs baseline: 1.0734x; 1.0734x over previous
import jax
import jax.numpy as jnp
from jax import lax
from jax.experimental import pallas as pl
from jax.experimental.pallas import tpu as pltpu
from jax.experimental.pallas import tpu_sc as plsc

B = 16384
DIM = 64
N = 1000000
NC = 2
NS = 16
NW = NC * NS
CPW = B // NW
NGRP = CPW // 16

NTC = (N + 127) // 128
NFULL = (N // 128) * 128
TC_LO = NTC // NW
TC_XTRA = NTC - TC_LO * NW
CHW = 768
NCHUNK = 42
WCAP = 1600
CCAP = 112
SENT = 2147483647

_LN2 = 0.6931471805599453
_SQRT2 = 1.4142135623730951


def _iota16():
    return lax.iota(jnp.int32, 16)


def _splat(x, dtype=jnp.int32):
    return jnp.zeros((16,), dtype) + x


def _ln(x):
    bits = plsc.bitcast(x, jnp.int32)
    ex = lax.shift_right_logical(bits, 23) - 127
    m = plsc.bitcast(
        jnp.bitwise_or(jnp.bitwise_and(bits, 0x7FFFFF), 0x3F800000),
        jnp.float32,
    )
    big = m > _SQRT2
    m = jnp.where(big, m * 0.5, m)
    exf = ex.astype(jnp.float32) + jnp.where(big, 1.0, 0.0)
    z = (m - 1.0) / (m + 1.0)
    z2 = z * z
    p = 1.0 + z2 * (1.0 / 3.0 + z2 * (1.0 / 5.0 + z2 * (1.0 / 7.0)))
    return exf * _LN2 + 2.0 * z * p


def _k1_body(et_h, left_h, right_h, rows_h,
             piece, wl_i, wl_o, cbuf0, cbuf1, tailbuf,
             ci0, co0, ci1, co1, stage0, stage1, semf0, semf1, sems):
    c = lax.axis_index("c")
    s = lax.axis_index("s")
    wid = s * NC + c
    ts = wid * TC_LO + jnp.minimum(wid, TC_XTRA)
    te = ts + TC_LO + jnp.where(wid < TC_XTRA, 1, 0)
    rs = ts * 128
    re = jnp.minimum(te * 128, N)
    rem_end = jnp.minimum(re, NFULL)

    def _fetch_desc(g, buf, semf):
        fb = pl.multiple_of(
            jnp.minimum(rs + g * CHW, rem_end - CHW), 128)
        return pltpu.make_async_copy(et_h.at[:, pl.ds(fb, CHW)], buf, semf)

    _fetch_desc(0, cbuf0, semf0).start()
    _fetch_desc(1, cbuf1, semf1).start()

    def _filter(src_h, side, cnt0):
        def piece_loop(p, cnt):
            pltpu.sync_copy(src_h.at[pl.ds(p * 2048, 2048)], piece)

            def t_loop(t, carry):
                cnt, offv = carry
                v = piece[pl.ds(t * 16, 16)]
                mask = (v >= rs) & (v < re)
                pos = plsc.cumsum(mask.astype(jnp.int32))
                idxs = pos + (cnt - 1)
                plsc.store_scatter(wl_i, [idxs], v, mask=mask)
                plsc.store_scatter(wl_o, [idxs], offv, mask=mask)
                return cnt + pos[15], offv + 16 * DIM
            offv0 = (_splat(side * B + p * 2048) + _iota16()) * DIM
            cnt, _ = lax.fori_loop(0, 128, t_loop, (cnt, offv0))
            return cnt
        return lax.fori_loop(0, 8, piece_loop, cnt0)

    m = _filter(left_h, 0, jnp.int32(0))
    m = _filter(right_h, 1, m)
    plsc.store_scatter(wl_i, [_iota16() + m], _splat(SENT))

    def _refilter(g, ci, co):
        lo = rs + g * CHW
        hi = jnp.minimum(lo + CHW, rem_end)
        return _refilter_win(lo, hi, ci, co)

    def _refilter_win(lo, hi, ci, co):
        trips = (m + 15) // 16

        def t_loop(t, mc):
            v = wl_i[pl.ds(t * 16, 16)]
            ov = wl_o[pl.ds(t * 16, 16)]
            mask = (v >= lo) & (v < hi)
            pos = plsc.cumsum(mask.astype(jnp.int32))
            idxs = pos + (mc - 1)
            plsc.store_scatter(ci, [idxs], v, mask=mask)
            plsc.store_scatter(co, [idxs], ov, mask=mask)
            return mc + pos[15]
        mc = lax.fori_loop(0, trips, t_loop, jnp.int32(0))
        c0 = ci[pl.ds(0, 16)]
        o0 = co[pl.ds(0, 16)]
        plsc.store_scatter(ci, [_iota16() + mc], _splat(c0[0]))
        plsc.store_scatter(co, [_iota16() + mc], _splat(o0[0]))
        return mc

    def _process(buf, g, mc, ci, co, stage):
        fb = pl.multiple_of(
            jnp.minimum(rs + g * CHW, rem_end - CHW), 128)
        ngr = (mc + 15) // 16

        def g_loop(gr, _):
            clv = ci[pl.ds(gr * 16, 16)] - fb
            bov = co[pl.ds(gr * 16, 16)]
            for i in range(16):
                cl = clv[i]
                slot = gr * 16 + i
                for q in range(4):
                    vq = plsc.load_gather(
                        buf, [_iota16() + 16 * q, _splat(cl)])
                    stage[pl.ds(slot * DIM + 16 * q, 16)] = vq
                pltpu.async_copy(
                    stage.at[pl.ds(slot * DIM, DIM)],
                    rows_h.at[pl.ds(pl.multiple_of(bov[i], DIM), DIM)],
                    sems)
            return 0
        lax.fori_loop(0, ngr, g_loop, 0)
        return ngr * 16

    def _drain(n):
        def d_loop(i, _):
            pltpu.make_async_copy(
                stage0.at[pl.ds(0, DIM)],
                rows_h.at[pl.ds(0, DIM)], sems).wait()
            return 0
        lax.fori_loop(0, n, d_loop, 0)

    mc_first = _refilter(0, ci0, co0)

    def outer(gg, carry):
        prev_n, mc0 = carry
        g0 = gg * 2
        mc1 = _refilter(g0 + 1, ci1, co1)
        _fetch_desc(g0, cbuf0, semf0).wait()
        n0 = _process(cbuf0, g0, mc0, ci0, co0, stage0)

        @pl.when(g0 + 2 < NCHUNK)
        def _():
            _fetch_desc(g0 + 2, cbuf0, semf0).start()
        _drain(prev_n)
        mc0n = _refilter(g0 + 2, ci0, co0)
        _fetch_desc(g0 + 1, cbuf1, semf1).wait()
        n1 = _process(cbuf1, g0 + 1, mc1, ci1, co1, stage1)

        @pl.when(g0 + 3 < NCHUNK)
        def _():
            _fetch_desc(g0 + 3, cbuf1, semf1).start()
        _drain(n0)
        return n1, mc0n

    last_n, _ = lax.fori_loop(
        0, NCHUNK // 2, outer, (jnp.int32(0), mc_first))
    _drain(last_n)

    @pl.when(wid == NW - 1)
    def _():
        pltpu.sync_copy(et_h.at[:, pl.ds(NFULL, N - NFULL)], tailbuf)
        mc = _refilter_win(NFULL, N, ci0, co0)
        ngr = (mc + 15) // 16

        def g_loop(gr, _):
            clv = ci0[pl.ds(gr * 16, 16)] - NFULL
            bov = co0[pl.ds(gr * 16, 16)]
            for i in range(16):
                cl = clv[i]
                slot = gr * 16 + i
                for q in range(4):
                    vq = plsc.load_gather(
                        tailbuf, [_iota16() + 16 * q, _splat(cl)])
                    stage0[pl.ds(slot * DIM + 16 * q, 16)] = vq
                pltpu.async_copy(
                    stage0.at[pl.ds(slot * DIM, DIM)],
                    rows_h.at[pl.ds(pl.multiple_of(bov[i], DIM), DIM)],
                    sems)
            return 0
        lax.fori_loop(0, ngr, g_loop, 0)
        _drain(ngr * 16)


def _k1(et, left, right):
    mesh = plsc.VectorSubcoreMesh(
        core_axis_name="c", subcore_axis_name="s",
        num_cores=NC, num_subcores=NS)
    call = pl.kernel(
        _k1_body,
        out_type=jax.ShapeDtypeStruct((2 * B * DIM,), jnp.float32),
        mesh=mesh,
        scratch_types=[
            pltpu.VMEM((2048,), jnp.int32),
            pltpu.VMEM((WCAP + 16,), jnp.int32),
            pltpu.VMEM((WCAP + 16,), jnp.int32),
            pltpu.VMEM((DIM, CHW), jnp.float32),
            pltpu.VMEM((DIM, CHW), jnp.float32),
            pltpu.VMEM((DIM, N - NFULL), jnp.float32),
            pltpu.VMEM((CCAP + 32,), jnp.int32),
            pltpu.VMEM((CCAP + 32,), jnp.int32),
            pltpu.VMEM((CCAP + 32,), jnp.int32),
            pltpu.VMEM((CCAP + 32,), jnp.int32),
            pltpu.VMEM(((CCAP + 16) * DIM,), jnp.float32),
            pltpu.VMEM(((CCAP + 16) * DIM,), jnp.float32),
            pltpu.SemaphoreType.DMA,
            pltpu.SemaphoreType.DMA,
            pltpu.SemaphoreType.DMA,
        ],
        compiler_params=pltpu.CompilerParams(
            needs_layout_passes=False, use_tc_tiling_on_sc=True),
    )
    return call(et, left, right)


def _k2_body(rows_h, bias_h, rew_h, right_h, out_h,
             ridx, lbuf, rbuf, bvals, rew, sbuf, outbuf, sem):
    c = lax.axis_index("c")
    s = lax.axis_index("s")
    wid = s * NC + c
    base = wid * CPW

    for j in range(CPW // 128):
        pltpu.sync_copy(right_h.at[pl.ds(base + j * 128, 128)], ridx.at[j])
    cps = [
        pltpu.async_copy(
            rows_h.at[pl.ds(base * DIM, CPW * DIM)], lbuf, sem),
        pltpu.async_copy(
            rows_h.at[pl.ds((B + base) * DIM, CPW * DIM)], rbuf, sem),
    ]
    for j in range(CPW // 128):
        cps.append(pltpu.async_copy(
            bias_h.at[ridx.at[j]], bvals.at[pl.ds(j * 128, 128)], sem))
    pltpu.sync_copy(rew_h.at[pl.ds(base, CPW)], rew)
    for cp in cps:
        cp.wait()

    def grp(g, acc):
        for i in range(16):
            r = g * 16 + i
            a = (lbuf[pl.ds(r * DIM, 16)] * rbuf[pl.ds(r * DIM, 16)]
                 + lbuf[pl.ds(r * DIM + 16, 16)] * rbuf[pl.ds(r * DIM + 16, 16)]
                 + lbuf[pl.ds(r * DIM + 32, 16)] * rbuf[pl.ds(r * DIM + 32, 16)]
                 + lbuf[pl.ds(r * DIM + 48, 16)] * rbuf[pl.ds(r * DIM + 48, 16)])
            sbuf[pl.ds(i * 16, 16)] = a
        rowbase = _iota16() * 16
        sc = jnp.zeros((16,), jnp.float32)
        for j in range(16):
            sc = sc + plsc.load_gather(sbuf, [rowbase + j])
        score = sc + bvals[pl.ds(g * 16, 16)]
        prob = 1.0 / (1.0 + jnp.exp(-score))
        prob = jnp.minimum(jnp.maximum(prob, 1e-5), 1.0 - 1e-5)
        return acc + _ln(prob) * _ln(1.0 - rew[pl.ds(g * 16, 16)])

    acc = lax.fori_loop(0, NGRP, grp, jnp.zeros((16,), jnp.float32))
    outbuf[...] = acc
    pltpu.sync_copy(outbuf, out_h.at[wid])


def _k2(rows, bias, reward, right):
    mesh = plsc.VectorSubcoreMesh(
        core_axis_name="c", subcore_axis_name="s",
        num_cores=NC, num_subcores=NS)
    call = pl.kernel(
        _k2_body,
        out_type=jax.ShapeDtypeStruct((NW, 16), jnp.float32),
        mesh=mesh,
        scratch_types=[
            pltpu.VMEM((CPW // 128, 128), jnp.int32),
            pltpu.VMEM((CPW * DIM,), jnp.float32),
            pltpu.VMEM((CPW * DIM,), jnp.float32),
            pltpu.VMEM((CPW,), jnp.float32),
            pltpu.VMEM((CPW,), jnp.float32),
            pltpu.VMEM((256,), jnp.float32),
            pltpu.VMEM((16,), jnp.float32),
            pltpu.SemaphoreType.DMA,
        ],
        compiler_params=pltpu.CompilerParams(
            needs_layout_passes=False, use_tc_tiling_on_sc=False),
    )
    return call(rows, bias, reward, right)


@jax.jit
def kernel(emb, bias, reward, left, right):
    left = left.astype(jnp.int32)
    right = right.astype(jnp.int32)
    rows = _k1(emb.T, left, right)
    part = _k2(rows, bias, reward, right)
    return jnp.sum(part)


# speedup vs baseline: 1.1087x; 1.0329x over previous
import jax
import jax.numpy as jnp
from jax import lax
from jax.experimental import pallas as pl
from jax.experimental.pallas import tpu as pltpu
from jax.experimental.pallas import tpu_sc as plsc

B = 16384
DIM = 64
N = 1000000
NC = 2
NS = 16
NW = NC * NS
CPW = B // NW
NGRP = CPW // 16

NTC = (N + 127) // 128
NFULL = (N // 128) * 128
TC_LO = NTC // NW
TC_XTRA = NTC - TC_LO * NW
CHW = 768
NCHUNK = 42
WCAP = 1300
CCAP = 96
NBK = 6
BKW = 7 * CHW
BCAP = 288
SENT = 2147483647

_LN2 = 0.6931471805599453
_SQRT2 = 1.4142135623730951


def _iota16():
    return lax.iota(jnp.int32, 16)


def _splat(x, dtype=jnp.int32):
    return jnp.zeros((16,), dtype) + x


def _ln(x):
    bits = plsc.bitcast(x, jnp.int32)
    ex = lax.shift_right_logical(bits, 23) - 127
    m = plsc.bitcast(
        jnp.bitwise_or(jnp.bitwise_and(bits, 0x7FFFFF), 0x3F800000),
        jnp.float32,
    )
    big = m > _SQRT2
    m = jnp.where(big, m * 0.5, m)
    exf = ex.astype(jnp.float32) + jnp.where(big, 1.0, 0.0)
    z = (m - 1.0) / (m + 1.0)
    z2 = z * z
    p = 1.0 + z2 * (1.0 / 3.0 + z2 * (1.0 / 5.0 + z2 * (1.0 / 7.0)))
    return exf * _LN2 + 2.0 * z * p


def _k1_body(et_h, left_h, right_h, rows_h,
             piece, wl_i, wl_o, bk_i, bk_o, cbuf0, cbuf1, tailbuf,
             ci0, co0, ci1, co1, stage0, stage1, semf0, semf1, sems):
    c = lax.axis_index("c")
    s = lax.axis_index("s")
    wid = s * NC + c
    ts = wid * TC_LO + jnp.minimum(wid, TC_XTRA)
    te = ts + TC_LO + jnp.where(wid < TC_XTRA, 1, 0)
    rs = ts * 128
    re = jnp.minimum(te * 128, N)
    rem_end = jnp.minimum(re, NFULL)

    def _fetch_desc(g, buf, semf):
        fb = pl.multiple_of(
            jnp.minimum(rs + g * CHW, rem_end - CHW), 128)
        return pltpu.make_async_copy(et_h.at[:, pl.ds(fb, CHW)], buf, semf)

    _fetch_desc(0, cbuf0, semf0).start()
    _fetch_desc(1, cbuf1, semf1).start()

    def _filter(src_h, side, cnt0):
        def piece_loop(p, cnt):
            pltpu.sync_copy(src_h.at[pl.ds(p * 1024, 1024)], piece)

            def t_loop(t, carry):
                cnt, offv = carry
                v = piece[pl.ds(t * 16, 16)]
                mask = (v >= rs) & (v < re)
                pos = plsc.cumsum(mask.astype(jnp.int32))
                idxs = pos + (cnt - 1)
                plsc.store_scatter(wl_i, [idxs], v, mask=mask)
                plsc.store_scatter(wl_o, [idxs], offv, mask=mask)
                return cnt + pos[15], offv + 16 * DIM
            offv0 = (_splat(side * B + p * 1024) + _iota16()) * DIM
            cnt, _ = lax.fori_loop(0, 64, t_loop, (cnt, offv0))
            return cnt
        return lax.fori_loop(0, 16, piece_loop, cnt0)

    m = _filter(left_h, 0, jnp.int32(0))
    m = _filter(right_h, 1, m)
    plsc.store_scatter(wl_i, [_iota16() + m], _splat(SENT))

    def bset(t, _):
        bk_i[pl.ds(t * 16, 16)] = _splat(SENT)
        return 0
    lax.fori_loop(0, NBK * BCAP // 16, bset, 0)

    def bucket_pass(t, carry):
        v = wl_i[pl.ds(t * 16, 16)]
        ov = wl_o[pl.ds(t * 16, 16)]
        new = []
        for q in range(NBK):
            cq = carry[q]
            maskq = (v >= rs + q * BKW) & (v < rs + (q + 1) * BKW)
            pos = plsc.cumsum(maskq.astype(jnp.int32))
            idxs = pos + (q * BCAP + cq - 1)
            plsc.store_scatter(bk_i, [idxs], v, mask=maskq)
            plsc.store_scatter(bk_o, [idxs], ov, mask=maskq)
            new.append(cq + pos[15])
        return tuple(new)
    lax.fori_loop(0, (m + 15) // 16, bucket_pass,
                  tuple(jnp.int32(0) for _ in range(NBK)))

    def _refilter(g, ci, co):
        lo = rs + g * CHW
        hi = jnp.minimum(lo + CHW, rem_end)
        qb = pl.multiple_of((g // 7) * BCAP, 16)
        trips = BCAP // 16

        def t_loop(t, mc):
            v = bk_i[pl.ds(qb + t * 16, 16)]
            ov = bk_o[pl.ds(qb + t * 16, 16)]
            mask = (v >= lo) & (v < hi)
            pos = plsc.cumsum(mask.astype(jnp.int32))
            idxs = pos + (mc - 1)
            plsc.store_scatter(ci, [idxs], v, mask=mask)
            plsc.store_scatter(co, [idxs], ov, mask=mask)
            return mc + pos[15]
        mc = lax.fori_loop(0, trips, t_loop, jnp.int32(0))
        c0 = ci[pl.ds(0, 16)]
        o0 = co[pl.ds(0, 16)]
        plsc.store_scatter(ci, [_iota16() + mc], _splat(c0[0]))
        plsc.store_scatter(co, [_iota16() + mc], _splat(o0[0]))
        return mc

    def _refilter_win(lo, hi, ci, co):
        trips = (m + 15) // 16

        def t_loop(t, mc):
            v = wl_i[pl.ds(t * 16, 16)]
            ov = wl_o[pl.ds(t * 16, 16)]
            mask = (v >= lo) & (v < hi)
            pos = plsc.cumsum(mask.astype(jnp.int32))
            idxs = pos + (mc - 1)
            plsc.store_scatter(ci, [idxs], v, mask=mask)
            plsc.store_scatter(co, [idxs], ov, mask=mask)
            return mc + pos[15]
        mc = lax.fori_loop(0, trips, t_loop, jnp.int32(0))
        c0 = ci[pl.ds(0, 16)]
        o0 = co[pl.ds(0, 16)]
        plsc.store_scatter(ci, [_iota16() + mc], _splat(c0[0]))
        plsc.store_scatter(co, [_iota16() + mc], _splat(o0[0]))
        return mc

    def _process(buf, g, mc, ci, co, stage):
        fb = pl.multiple_of(
            jnp.minimum(rs + g * CHW, rem_end - CHW), 128)
        ngr = (mc + 15) // 16

        def g_loop(gr, _):
            clv = ci[pl.ds(gr * 16, 16)] - fb
            bov = co[pl.ds(gr * 16, 16)]
            for i in range(16):
                cl = clv[i]
                slot = gr * 16 + i
                for q in range(4):
                    vq = plsc.load_gather(
                        buf, [_iota16() + 16 * q, _splat(cl)])
                    stage[pl.ds(slot * DIM + 16 * q, 16)] = vq
                pltpu.async_copy(
                    stage.at[pl.ds(slot * DIM, DIM)],
                    rows_h.at[pl.ds(pl.multiple_of(bov[i], DIM), DIM)],
                    sems)
            return 0
        lax.fori_loop(0, ngr, g_loop, 0)
        return ngr * 16

    def _drain(n):
        def d_loop(i, _):
            pltpu.make_async_copy(
                stage0.at[pl.ds(0, DIM)],
                rows_h.at[pl.ds(0, DIM)], sems).wait()
            return 0
        lax.fori_loop(0, n, d_loop, 0)

    mc_first = _refilter(0, ci0, co0)

    def outer(gg, carry):
        prev_n, mc0 = carry
        g0 = gg * 2
        mc1 = _refilter(g0 + 1, ci1, co1)
        _fetch_desc(g0, cbuf0, semf0).wait()
        n0 = _process(cbuf0, g0, mc0, ci0, co0, stage0)

        @pl.when(g0 + 2 < NCHUNK)
        def _():
            _fetch_desc(g0 + 2, cbuf0, semf0).start()
        _drain(prev_n)
        mc0n = _refilter(g0 + 2, ci0, co0)
        _fetch_desc(g0 + 1, cbuf1, semf1).wait()
        n1 = _process(cbuf1, g0 + 1, mc1, ci1, co1, stage1)

        @pl.when(g0 + 3 < NCHUNK)
        def _():
            _fetch_desc(g0 + 3, cbuf1, semf1).start()
        _drain(n0)
        return n1, mc0n

    last_n, _ = lax.fori_loop(
        0, NCHUNK // 2, outer, (jnp.int32(0), mc_first))
    _drain(last_n)

    @pl.when(wid == NW - 1)
    def _():
        pltpu.sync_copy(et_h.at[:, pl.ds(NFULL, N - NFULL)], tailbuf)
        mc = _refilter_win(NFULL, N, ci0, co0)
        ngr = (mc + 15) // 16

        def g_loop(gr, _):
            clv = ci0[pl.ds(gr * 16, 16)] - NFULL
            bov = co0[pl.ds(gr * 16, 16)]
            for i in range(16):
                cl = clv[i]
                slot = gr * 16 + i
                for q in range(4):
                    vq = plsc.load_gather(
                        tailbuf, [_iota16() + 16 * q, _splat(cl)])
                    stage0[pl.ds(slot * DIM + 16 * q, 16)] = vq
                pltpu.async_copy(
                    stage0.at[pl.ds(slot * DIM, DIM)],
                    rows_h.at[pl.ds(pl.multiple_of(bov[i], DIM), DIM)],
                    sems)
            return 0
        lax.fori_loop(0, ngr, g_loop, 0)
        _drain(ngr * 16)


def _k1(et, left, right):
    mesh = plsc.VectorSubcoreMesh(
        core_axis_name="c", subcore_axis_name="s",
        num_cores=NC, num_subcores=NS)
    call = pl.kernel(
        _k1_body,
        out_type=jax.ShapeDtypeStruct((2 * B * DIM,), jnp.float32),
        mesh=mesh,
        scratch_types=[
            pltpu.VMEM((1024,), jnp.int32),
            pltpu.VMEM((WCAP + 16,), jnp.int32),
            pltpu.VMEM((WCAP + 16,), jnp.int32),
            pltpu.VMEM((NBK * BCAP + 16,), jnp.int32),
            pltpu.VMEM((NBK * BCAP + 16,), jnp.int32),
            pltpu.VMEM((DIM, CHW), jnp.float32),
            pltpu.VMEM((DIM, CHW), jnp.float32),
            pltpu.VMEM((DIM, N - NFULL), jnp.float32),
            pltpu.VMEM((CCAP + 32,), jnp.int32),
            pltpu.VMEM((CCAP + 32,), jnp.int32),
            pltpu.VMEM((CCAP + 32,), jnp.int32),
            pltpu.VMEM((CCAP + 32,), jnp.int32),
            pltpu.VMEM(((CCAP + 16) * DIM,), jnp.float32),
            pltpu.VMEM(((CCAP + 16) * DIM,), jnp.float32),
            pltpu.SemaphoreType.DMA,
            pltpu.SemaphoreType.DMA,
            pltpu.SemaphoreType.DMA,
        ],
        compiler_params=pltpu.CompilerParams(
            needs_layout_passes=False, use_tc_tiling_on_sc=True),
    )
    return call(et, left, right)


def _k2_body(rows_h, bias_h, rew_h, right_h, out_h,
             ridx, lbuf, rbuf, bvals, rew, sbuf, outbuf, sem):
    c = lax.axis_index("c")
    s = lax.axis_index("s")
    wid = s * NC + c
    base = wid * CPW

    for j in range(CPW // 128):
        pltpu.sync_copy(right_h.at[pl.ds(base + j * 128, 128)], ridx.at[j])
    cps = [
        pltpu.async_copy(
            rows_h.at[pl.ds(base * DIM, CPW * DIM)], lbuf, sem),
        pltpu.async_copy(
            rows_h.at[pl.ds((B + base) * DIM, CPW * DIM)], rbuf, sem),
    ]
    for j in range(CPW // 128):
        cps.append(pltpu.async_copy(
            bias_h.at[ridx.at[j]], bvals.at[pl.ds(j * 128, 128)], sem))
    pltpu.sync_copy(rew_h.at[pl.ds(base, CPW)], rew)
    for cp in cps:
        cp.wait()

    def grp(g, acc):
        for i in range(16):
            r = g * 16 + i
            a = (lbuf[pl.ds(r * DIM, 16)] * rbuf[pl.ds(r * DIM, 16)]
                 + lbuf[pl.ds(r * DIM + 16, 16)] * rbuf[pl.ds(r * DIM + 16, 16)]
                 + lbuf[pl.ds(r * DIM + 32, 16)] * rbuf[pl.ds(r * DIM + 32, 16)]
                 + lbuf[pl.ds(r * DIM + 48, 16)] * rbuf[pl.ds(r * DIM + 48, 16)])
            sbuf[pl.ds(i * 16, 16)] = a
        rowbase = _iota16() * 16
        sc = jnp.zeros((16,), jnp.float32)
        for j in range(16):
            sc = sc + plsc.load_gather(sbuf, [rowbase + j])
        score = sc + bvals[pl.ds(g * 16, 16)]
        prob = 1.0 / (1.0 + jnp.exp(-score))
        prob = jnp.minimum(jnp.maximum(prob, 1e-5), 1.0 - 1e-5)
        return acc + _ln(prob) * _ln(1.0 - rew[pl.ds(g * 16, 16)])

    acc = lax.fori_loop(0, NGRP, grp, jnp.zeros((16,), jnp.float32))
    outbuf[...] = acc
    pltpu.sync_copy(outbuf, out_h.at[wid])


def _k2(rows, bias, reward, right):
    mesh = plsc.VectorSubcoreMesh(
        core_axis_name="c", subcore_axis_name="s",
        num_cores=NC, num_subcores=NS)
    call = pl.kernel(
        _k2_body,
        out_type=jax.ShapeDtypeStruct((NW, 16), jnp.float32),
        mesh=mesh,
        scratch_types=[
            pltpu.VMEM((CPW // 128, 128), jnp.int32),
            pltpu.VMEM((CPW * DIM,), jnp.float32),
            pltpu.VMEM((CPW * DIM,), jnp.float32),
            pltpu.VMEM((CPW,), jnp.float32),
            pltpu.VMEM((CPW,), jnp.float32),
            pltpu.VMEM((256,), jnp.float32),
            pltpu.VMEM((16,), jnp.float32),
            pltpu.SemaphoreType.DMA,
        ],
        compiler_params=pltpu.CompilerParams(
            needs_layout_passes=False, use_tc_tiling_on_sc=False),
    )
    return call(rows, bias, reward, right)


@jax.jit
def kernel(emb, bias, reward, left, right):
    left = left.astype(jnp.int32)
    right = right.astype(jnp.int32)
    rows = _k1(emb.T, left, right)
    part = _k2(rows, bias, reward, right)
    return jnp.sum(part)
